# baseline (speedup 1.0000x reference)
import functools
import math

import jax
import jax.numpy as jnp
from jax import lax
from jax.experimental import pallas as pl
from jax.experimental.pallas import tpu as pltpu
from jax.experimental.pallas import tpu_sc as plsc

N = 10000
E = 320000
H = 64
FF = 128
L = 2
ED = 4

N_PAD = 10240
NW = 32
C = 128
NCHUNK = 80
PAIRS = NCHUNK // 2
CW = C * NCHUNK
E_PAD = CW * NW
ROWS_PER_SUB = N_PAD // 16
ACHUNK = E_PAD // 16 // C
C1 = 64
NCHUNK1 = CW // C1
PAIRS1 = NCHUNK1 // 2
NCH_A = 56
NCH_B = 104

BN = 2048


def _mm_bias_kernel(x_ref, w_ref, b_ref, o_ref):
    o_ref[...] = (
        jnp.dot(x_ref[...], w_ref[...], preferred_element_type=jnp.float32, precision=lax.Precision.HIGHEST)
        + b_ref[...]
    )


def _tc_matmul_bias(x, w, b):
    n, k = x.shape
    m = w.shape[1]
    return pl.pallas_call(
        _mm_bias_kernel,
        grid=(n // BN,),
        in_specs=[
            pl.BlockSpec((BN, k), lambda i: (i, 0)),
            pl.BlockSpec((k, m), lambda i: (0, 0)),
            pl.BlockSpec((1, m), lambda i: (0, 0)),
        ],
        out_specs=pl.BlockSpec((BN, m), lambda i: (i, 0)),
        out_shape=jax.ShapeDtypeStruct((n, m), jnp.float32),
    )(x, w, b)


def _qkv_kernel(z_ref, w_ref, q_ref, k_ref, v_ref):
    h = jnp.dot(z_ref[...], w_ref[...], preferred_element_type=jnp.float32, precision=lax.Precision.HIGHEST)
    q_ref[...] = h[:, : 2 * H].astype(jnp.bfloat16)
    k_ref[...] = h[:, 2 * H : 4 * H].astype(jnp.bfloat16)
    v_ref[...] = h[:, 4 * H :]


def _tc_qkv(z, wqkv):
    return pl.pallas_call(
        _qkv_kernel,
        grid=(N_PAD // BN,),
        in_specs=[
            pl.BlockSpec((BN, 2 * H), lambda i: (i, 0)),
            pl.BlockSpec((2 * H, 6 * H), lambda i: (0, 0)),
        ],
        out_specs=[
            pl.BlockSpec((BN, 2 * H), lambda i: (i, 0)),
            pl.BlockSpec((BN, 2 * H), lambda i: (i, 0)),
            pl.BlockSpec((BN, 2 * H), lambda i: (i, 0)),
        ],
        out_shape=[jax.ShapeDtypeStruct((N_PAD, 2 * H), jnp.bfloat16),
                   jax.ShapeDtypeStruct((N_PAD, 2 * H), jnp.bfloat16),
                   jax.ShapeDtypeStruct((N_PAD, 2 * H), jnp.float32)],
    )(z, wqkv)


def _score_kernel(p_ref, ea_ref, b_ref, we_ref, o_ref):
    s = (jnp.dot(p_ref[...], b_ref[...], preferred_element_type=jnp.float32, precision=lax.Precision.HIGHEST)
         + jnp.dot(ea_ref[...], we_ref[...],
                   preferred_element_type=jnp.float32, precision=lax.Precision.HIGHEST))
    o_ref[...] = jnp.exp(s)


def _tc_score(pflat, eaflat, bsel, wesel):
    blk = 4096
    rows = E_PAD // 8
    return pl.pallas_call(
        _score_kernel,
        grid=(rows // blk,),
        in_specs=[
            pl.BlockSpec((blk, 128), lambda i: (i, 0)),
            pl.BlockSpec((blk, 8 * ED), lambda i: (i, 0)),
            pl.BlockSpec((128, 8), lambda i: (0, 0)),
            pl.BlockSpec((8 * ED, 8), lambda i: (0, 0)),
        ],
        out_specs=pl.BlockSpec((blk, 8), lambda i: (i, 0)),
        out_shape=jax.ShapeDtypeStruct((rows, 8), jnp.float32),
    )(pflat, eaflat, bsel, wesel)


def _ffn_kernel(z_ref, a_ref, w1_ref, b1_ref, bm_ref, w2_ref, b2_ref, o_ref):
    za = (z_ref[...] + a_ref[0].astype(jnp.float32)
          + a_ref[1].astype(jnp.float32))
    h = jnp.dot(za, w1_ref[...], preferred_element_type=jnp.float32, precision=lax.Precision.HIGHEST) + b1_ref[...]
    hr = h[:, :FF]
    hi = h[:, FF:]
    mag = jnp.sqrt(hr * hr + hi * hi + 1e-6)
    s = jnp.maximum(mag + bm_ref[...], 0.0) / mag
    hs = jnp.concatenate([hr * s, hi * s], axis=1)
    f = jnp.dot(hs, w2_ref[...], preferred_element_type=jnp.float32, precision=lax.Precision.HIGHEST) + b2_ref[...]
    o_ref[...] = f + za


def _tc_ffn(z, a2, w1, b1, bm, w2, b2):
    return pl.pallas_call(
        _ffn_kernel,
        grid=(N_PAD // BN,),
        in_specs=[
            pl.BlockSpec((BN, 2 * H), lambda i: (i, 0)),
            pl.BlockSpec((2, BN, 2 * H), lambda i: (0, i, 0)),
            pl.BlockSpec((2 * H, 2 * FF), lambda i: (0, 0)),
            pl.BlockSpec((1, 2 * FF), lambda i: (0, 0)),
            pl.BlockSpec((1, FF), lambda i: (0, 0)),
            pl.BlockSpec((2 * FF, 2 * H), lambda i: (0, 0)),
            pl.BlockSpec((1, 2 * H), lambda i: (0, 0)),
        ],
        out_specs=pl.BlockSpec((BN, 2 * H), lambda i: (i, 0)),
        out_shape=jax.ShapeDtypeStruct((N_PAD, 2 * H), jnp.float32),
    )(z, a2, w1, b1, bm, w2, b2)


def _final_kernel(z_ref, wv_ref, ob_ref, o_ref, acc_ref):
    i = pl.program_id(0)

    @pl.when(i == 0)
    def _init():
        acc_ref[...] = jnp.zeros_like(acc_ref)

    z = z_ref[...]
    zr = z[:, :H]
    zi = z[:, H:]
    mz = jnp.sqrt(zr * zr + zi * zi + 1e-6)
    row = i * BN + lax.broadcasted_iota(jnp.int32, (BN, H), 0)
    mz = jnp.where(row < N, mz, 0.0)
    part = jnp.sum(mz, axis=0, keepdims=True)
    partp = jnp.concatenate([part, jnp.zeros((1, H), jnp.float32)], axis=1)
    acc_ref[0:1, :] = acc_ref[0:1, :] + partp

    o_ref[...] = jnp.zeros((8, 128), jnp.float32)

    @pl.when(i == pl.num_programs(0) - 1)
    def _fin():
        tot = jnp.sum(acc_ref[0:1, :] * wv_ref[...])
        outv = tot + float(N) * ob_ref[0, 0]
        ri = lax.broadcasted_iota(jnp.int32, (8, 128), 0)
        ci = lax.broadcasted_iota(jnp.int32, (8, 128), 1)
        o_ref[...] = jnp.where((ri == 0) & (ci == 0), outv, 0.0)


def _tc_final(z, wvec, obvec):
    return pl.pallas_call(
        _final_kernel,
        grid=(N_PAD // BN,),
        in_specs=[
            pl.BlockSpec((BN, 2 * H), lambda i: (i, 0)),
            pl.BlockSpec((1, 128), lambda i: (0, 0)),
            pl.BlockSpec((1, 128), lambda i: (0, 0)),
        ],
        out_specs=pl.BlockSpec((8, 128), lambda i: (0, 0)),
        out_shape=jax.ShapeDtypeStruct((8, 128), jnp.float32),
        scratch_shapes=[pltpu.VMEM((8, 128), jnp.float32)],
    )(z, wvec, obvec)


def _mesh():
    return plsc.VectorSubcoreMesh(
        core_axis_name="c", subcore_axis_name="s", num_cores=2, num_subcores=16
    )


_SC_PARAMS = pltpu.CompilerParams(needs_layout_passes=False)


def _pass1_body(qk_hbm, src_hbm, dst_hbm,
                p_out,
                sidx0, sidx1, didx0, didx1,
                qbuf0, qbuf1, kbuf0, kbuf1, pbuf0, pbuf1,
                qk_sh,
                s_si0, s_si1, s_di0, s_di1, s_q0, s_q1, s_k0, s_k1,
                s_pw0, s_pw1):
    sidx = [sidx0, sidx1]
    didx = [didx0, didx1]
    qbuf = [qbuf0, qbuf1]
    kbuf = [kbuf0, kbuf1]
    pbuf = [pbuf0, pbuf1]
    s_si = [s_si0, s_si1]
    s_di = [s_di0, s_di1]
    s_q = [s_q0, s_q1]
    s_k = [s_k0, s_k1]
    s_pw = [s_pw0, s_pw1]

    cid = lax.axis_index("c")
    sid = lax.axis_index("s")
    wid = sid * 2 + cid
    base0 = wid * CW

    pltpu.sync_copy(qk_hbm.at[pl.ds(sid * ROWS_PER_SUB, ROWS_PER_SUB)],
                    qk_sh.at[pl.ds(sid * ROWS_PER_SUB, ROWS_PER_SUB)])
    plsc.subcore_barrier()

    def issue_idx(t, b):
        base = pl.multiple_of(base0 + t * C1, C1)
        pltpu.async_copy(src_hbm.at[pl.ds(base, C1)], sidx[b], s_si[b])
        pltpu.async_copy(dst_hbm.at[pl.ds(base, C1)], didx[b], s_di[b])

    def wait_idx(b):
        pltpu.make_async_copy(src_hbm.at[pl.ds(0, C1)], sidx[b], s_si[b]).wait()
        pltpu.make_async_copy(dst_hbm.at[pl.ds(0, C1)], didx[b], s_di[b]).wait()

    def issue_gather(b):
        pltpu.async_copy(qk_sh.at[didx[b]], qbuf[b], s_q[b])
        pltpu.async_copy(qk_sh.at[sidx[b]], kbuf[b], s_k[b])

    def wait_gather(b):
        pltpu.make_async_copy(qk_sh.at[didx[b]], qbuf[b], s_q[b]).wait()
        pltpu.make_async_copy(qk_sh.at[sidx[b]], kbuf[b], s_k[b]).wait()

    def compute(t, b):
        @pl.when(t < NCHUNK1 - 2)
        def _pref():
            issue_idx(t + 2, b)

        @pl.when(t >= 2)
        def _wb():
            pltpu.make_async_copy(pbuf[b], p_out.at[pl.ds(0, C1 // 8)],
                                  s_pw[b]).wait()

        qb = qbuf[b]
        kb = kbuf[b]
        pb = pbuf[b]

        def _group(g, gcarry):
            for e in range(16):
                row = g * 16 + e
                acc0 = jnp.zeros((16,), jnp.float32)
                acc1 = jnp.zeros((16,), jnp.float32)
                for j in range(4):
                    qv = plsc.bitcast(qb[row, pl.ds(j * 16, 16)],
                                      jnp.bfloat16)
                    kv = plsc.bitcast(kb[row, pl.ds(64 + j * 16, 16)],
                                      jnp.bfloat16)
                    pa, pbv = plsc.unpack(qv * kv,
                                          format=plsc.PackFormat.INTERLEAVED)
                    acc0 = acc0 + pa
                    acc1 = acc1 + pbv
                pb[2 * g + e // 8, pl.ds((e % 8) * 16, 16)] = acc0 + acc1
            return gcarry

        lax.fori_loop(0, C1 // 16, _group, 0)
        base = pl.multiple_of(base0 + t * C1, C1)
        pltpu.async_copy(
            pb,
            p_out.at[pl.ds(pl.multiple_of(base // 8, C1 // 8), C1 // 8)],
            s_pw[b])

    issue_idx(0, 0)
    issue_idx(1, 1)
    wait_idx(0)
    issue_gather(0)

    def _pair(p, carry):
        t0 = 2 * p
        wait_idx(1)
        issue_gather(1)
        wait_gather(0)
        compute(t0, 0)

        @pl.when(p < PAIRS1 - 1)
        def _nxt():
            wait_idx(0)
            issue_gather(0)

        wait_gather(1)
        compute(t0 + 1, 1)
        return carry

    lax.fori_loop(0, PAIRS1, _pair, 0)
    pltpu.make_async_copy(pbuf[0], p_out.at[pl.ds(0, C1 // 8)], s_pw[0]).wait()
    pltpu.make_async_copy(pbuf[1], p_out.at[pl.ds(0, C1 // 8)], s_pw[1]).wait()


def _sc_pass1(qk, src, dst):
    fn = pl.kernel(
        _pass1_body,
        out_type=[
            jax.ShapeDtypeStruct((E_PAD // 8, 128), jnp.float32),
        ],
        mesh=_mesh(),
        scratch_types=[
            pltpu.VMEM((C1,), jnp.int32),
            pltpu.VMEM((C1,), jnp.int32),
            pltpu.VMEM((C1,), jnp.int32),
            pltpu.VMEM((C1,), jnp.int32),
            pltpu.VMEM((C1, 128), jnp.int32),
            pltpu.VMEM((C1, 128), jnp.int32),
            pltpu.VMEM((C1, 128), jnp.int32),
            pltpu.VMEM((C1, 128), jnp.int32),
            pltpu.VMEM((C1 // 8, 128), jnp.float32),
            pltpu.VMEM((C1 // 8, 128), jnp.float32),
            pltpu.VMEM_SHARED((N_PAD, 128), jnp.int32),
        ] + [pltpu.SemaphoreType.DMA] * 10,
        compiler_params=_SC_PARAMS,
    )
    return fn(qk, src, dst)[0]


def _pass2_body(v_hbm, ex_hbm, src_hbm, dst_hbm,
                a_out,
                sidx0, sidx1, didx0, didx1, sdidx0, sdidx1,
                adx0, adx1, aex0, aex1,
                vbuf0, vbuf1, exbuf0, exbuf1, dnb0, dnb1,
                zbuf, zfbuf, den_sh, a_sh,
                s_si0, s_si1, s_di0, s_di1, s_v0, s_v1, s_x0, s_x1,
                s_d0, s_d1, s_sc0, s_sc1,
                s_ai0, s_ai1, s_ax0, s_ax1, s_as0, s_as1):
    sidx = [sidx0, sidx1]
    didx = [didx0, didx1]
    sdidx = [sdidx0, sdidx1]
    adx = [adx0, adx1]
    aex = [aex0, aex1]
    vbuf = [vbuf0, vbuf1]
    exbuf = [exbuf0, exbuf1]
    dnb = [dnb0, dnb1]
    s_si = [s_si0, s_si1]
    s_di = [s_di0, s_di1]
    s_v = [s_v0, s_v1]
    s_x = [s_x0, s_x1]
    s_d = [s_d0, s_d1]
    s_sc = [s_sc0, s_sc1]
    s_ai = [s_ai0, s_ai1]
    s_ax = [s_ax0, s_ax1]
    s_as = [s_as0, s_as1]

    cid = lax.axis_index("c")
    sid = lax.axis_index("s")
    nch = jnp.where(cid == 0, NCH_A, NCH_B)
    npair = nch // 2
    base0 = sid * (2 * CW) + cid * (NCH_A * C)

    for r in range(16):
        for j in range(8):
            zbuf[r, pl.ds(j * 16, 16)] = jnp.zeros((16,), jnp.float32)
    for j in range(8):
        zfbuf[pl.ds(j * 16, 16)] = jnp.zeros((16,), jnp.float32)

    def _zrow(r, carry):
        pltpu.sync_copy(zbuf, a_sh.at[pl.ds(sid * ROWS_PER_SUB + r * 16, 16)])
        return carry

    lax.fori_loop(0, ROWS_PER_SUB // 16, _zrow, 0)

    def _zden(r, carry):
        pltpu.sync_copy(zfbuf,
                        den_sh.at[pl.ds(sid * ROWS_PER_SUB + r * 128, 128)])
        return carry

    lax.fori_loop(0, ROWS_PER_SUB // 128, _zden, 0)
    plsc.subcore_barrier()

    abase0 = sid * (E_PAD // 16)

    def a_issue_idx(t, b):
        base = pl.multiple_of(abase0 + t * C, C)
        pltpu.async_copy(dst_hbm.at[pl.ds(base, C)], adx[b], s_ai[b])
        pltpu.async_copy(ex_hbm.at[pl.ds(base, C)], aex[b], s_ax[b])

    def a_wait_idx(b):
        pltpu.make_async_copy(dst_hbm.at[pl.ds(0, C)], adx[b], s_ai[b]).wait()
        pltpu.make_async_copy(ex_hbm.at[pl.ds(0, C)], aex[b], s_ax[b]).wait()

    def a_issue_sc(b):
        pltpu.async_copy(aex[b], den_sh.at[adx[b]], s_as[b], add=True)

    def a_wait_sc(b):
        pltpu.make_async_copy(aex[b], den_sh.at[adx[b]], s_as[b]).wait()

    a_issue_idx(0, 0)
    a_issue_idx(1, 1)

    def _apair(p, carry):
        t0 = 2 * p
        a_wait_idx(0)
        a_issue_sc(0)
        a_wait_idx(1)
        a_issue_sc(1)
        a_wait_sc(0)

        @pl.when(p < ACHUNK // 2 - 1)
        def _n0():
            a_issue_idx(t0 + 2, 0)

        a_wait_sc(1)

        @pl.when(p < ACHUNK // 2 - 1)
        def _n1():
            a_issue_idx(t0 + 3, 1)

        return carry

    lax.fori_loop(0, ACHUNK // 2, _apair, 0)
    plsc.subcore_barrier()

    def issue_idx(t, b):
        base = pl.multiple_of(base0 + t * C, C)
        pltpu.async_copy(src_hbm.at[pl.ds(base, C)], sidx[b], s_si[b])
        pltpu.async_copy(dst_hbm.at[pl.ds(base, C)], didx[b], s_di[b])

    def wait_idx(b):
        pltpu.make_async_copy(src_hbm.at[pl.ds(0, C)], sidx[b], s_si[b]).wait()
        pltpu.make_async_copy(dst_hbm.at[pl.ds(0, C)], didx[b], s_di[b]).wait()

    def issue_gather(t, b):
        base = pl.multiple_of(base0 + t * C, C)
        pltpu.async_copy(v_hbm.at[sidx[b]], vbuf[b], s_v[b])
        pltpu.async_copy(ex_hbm.at[pl.ds(base, C)], exbuf[b], s_x[b])
        pltpu.async_copy(den_sh.at[didx[b]], dnb[b], s_d[b])

    def wait_gather(b):
        pltpu.make_async_copy(v_hbm.at[sidx[b]], vbuf[b], s_v[b]).wait()
        pltpu.make_async_copy(ex_hbm.at[pl.ds(0, C)], exbuf[b], s_x[b]).wait()
        pltpu.make_async_copy(den_sh.at[didx[b]], dnb[b], s_d[b]).wait()

    def issue_scatter(b):
        pltpu.async_copy(vbuf[b], a_sh.at[sdidx[b]], s_sc[b], add=True)

    def wait_scatter(b):
        pltpu.make_async_copy(vbuf[b], a_sh.at[sdidx[b]], s_sc[b]).wait()

    def compute(t, b):
        for gg in range(C // 16):
            sdidx[b][pl.ds(gg * 16, 16)] = didx[b][pl.ds(gg * 16, 16)]

        @pl.when(t < nch - 2)
        def _pref():
            issue_idx(t + 2, b)

        vb = vbuf[b]
        exb = exbuf[b]
        dn = dnb[b]

        def _group(g, gcarry):
            ex16 = exb[pl.ds(g * 16, 16)]
            den16 = dn[pl.ds(g * 16, 16)]
            alpha16 = ex16 / (den16 + 1e-9)
            for e in range(16):
                row = g * 16 + e
                av = jnp.full((16,), alpha16[e])
                for j in range(8):
                    vb[row, pl.ds(j * 16, 16)] = (
                        vb[row, pl.ds(j * 16, 16)] * av
                    )
            return gcarry

        lax.fori_loop(0, C // 16, _group, 0)

    issue_idx(0, 0)
    issue_idx(1, 1)
    wait_idx(0)
    issue_gather(0, 0)

    def _pair(p, carry):
        t0 = 2 * p

        @pl.when(p > 0)
        def _ws1():
            wait_scatter(1)

        wait_idx(1)
        issue_gather(t0 + 1, 1)
        wait_gather(0)
        compute(t0, 0)
        issue_scatter(0)

        @pl.when(p < npair - 1)
        def _nxt():
            wait_scatter(0)
            wait_idx(0)
            issue_gather(t0 + 2, 0)

        wait_gather(1)
        compute(t0 + 1, 1)
        issue_scatter(1)
        return carry

    lax.fori_loop(0, npair, _pair, 0)
    wait_scatter(0)
    wait_scatter(1)
    plsc.subcore_barrier()
    pltpu.sync_copy(
        a_sh.at[pl.ds(sid * ROWS_PER_SUB, ROWS_PER_SUB)],
        a_out.at[cid, pl.ds(sid * ROWS_PER_SUB, ROWS_PER_SUB)],
    )


def _sc_pass2(v, ex, src, dst):
    fn = pl.kernel(
        _pass2_body,
        out_type=[
            jax.ShapeDtypeStruct((2, N_PAD, 2 * H), jnp.float32),
        ],
        mesh=_mesh(),
        scratch_types=[
            pltpu.VMEM((C,), jnp.int32),
            pltpu.VMEM((C,), jnp.int32),
            pltpu.VMEM((C,), jnp.int32),
            pltpu.VMEM((C,), jnp.int32),
            pltpu.VMEM((C,), jnp.int32),
            pltpu.VMEM((C,), jnp.int32),
            pltpu.VMEM((C,), jnp.int32),
            pltpu.VMEM((C,), jnp.int32),
            pltpu.VMEM((C,), jnp.float32),
            pltpu.VMEM((C,), jnp.float32),
            pltpu.VMEM((C, 2 * H), jnp.float32),
            pltpu.VMEM((C, 2 * H), jnp.float32),
            pltpu.VMEM((C,), jnp.float32),
            pltpu.VMEM((C,), jnp.float32),
            pltpu.VMEM((C,), jnp.float32),
            pltpu.VMEM((C,), jnp.float32),
            pltpu.VMEM((16, 2 * H), jnp.float32),
            pltpu.VMEM((128,), jnp.float32),
            pltpu.VMEM_SHARED((N_PAD,), jnp.float32),
            pltpu.VMEM_SHARED((N_PAD, 2 * H), jnp.float32),
        ] + [pltpu.SemaphoreType.DMA] * 18,
        compiler_params=_SC_PARAMS,
    )
    return fn(v, ex, src, dst)[0]


def _cplx(wr, wi):
    return jnp.concatenate(
        [jnp.concatenate([wr, wi], axis=1),
         jnp.concatenate([-wi, wr], axis=1)],
        axis=0,
    )


def kernel(atom_types, coords_spherical, edge_index, edge_attr,
           emb_Wr, emb_Wi, emb_br, emb_bi,
           Wq_r, Wq_i, Wk_r, Wk_i, Wv_r, Wv_i, we,
           W1r, W1i, b1r, b1i, b_mod, W2r, W2i, b2r, b2i,
           out_W, out_b):
    f32 = jnp.float32
    d_in = atom_types.shape[1] + 3
    k_pad = 256

    x = jnp.concatenate([atom_types, coords_spherical], axis=1)
    x_pad = jnp.zeros((N_PAD, k_pad), f32).at[:N, :d_in].set(x)
    w_emb = jnp.zeros((k_pad, 2 * H), f32).at[:d_in].set(
        jnp.concatenate([emb_Wr, emb_Wi], axis=1))
    b_emb = jnp.concatenate([emb_br, emb_bi]).reshape(1, 2 * H)

    pad_e = E_PAD - E
    pad_idx = jnp.full((pad_e,), N_PAD - 1, jnp.int32)
    src_pad = jnp.concatenate([edge_index[0].astype(jnp.int32), pad_idx])
    dst_pad = jnp.concatenate([edge_index[1].astype(jnp.int32), pad_idx])
    ea_flat = jnp.concatenate(
        [edge_attr.astype(f32), jnp.zeros((pad_e, ED), f32)], axis=0
    ).reshape(E_PAD // 8, 8 * ED)

    bsel = (lax.broadcasted_iota(jnp.int32, (128, 8), 0) // 16
            == lax.broadcasted_iota(jnp.int32, (128, 8), 1)).astype(f32)

    z = _tc_matmul_bias(x_pad, w_emb, b_emb)

    inv_sqrt_h = 1.0 / math.sqrt(float(H))
    for i in range(L):
        wqkv = jnp.concatenate(
            [_cplx(Wq_r[i], Wq_i[i]) * inv_sqrt_h,
             _cplx(Wk_r[i], Wk_i[i]),
             _cplx(Wv_r[i], Wv_i[i])],
            axis=1,
        )
        q, k, v = _tc_qkv(z, wqkv)
        qk32 = lax.bitcast_convert_type(
            jnp.concatenate([q, k], axis=1).reshape(N_PAD, 128, 2), jnp.int32)

        wesel = ((lax.broadcasted_iota(jnp.int32, (8 * ED, 8), 0) // ED
                  == lax.broadcasted_iota(jnp.int32, (8 * ED, 8), 1))
                 .astype(f32)
                 * jnp.tile(we[i].astype(f32), 8)[:, None])

        pflat = _sc_pass1(qk32, src_pad, dst_pad)
        ex = _tc_score(pflat, ea_flat, bsel, wesel).reshape(E_PAD)
        a2 = _sc_pass2(v, ex, src_pad, dst_pad)

        w1 = _cplx(W1r[i], W1i[i])
        b1 = jnp.concatenate([b1r[i], b1i[i]]).reshape(1, 2 * FF)
        bm = b_mod[i].reshape(1, FF)
        w2 = _cplx(W2r[i], W2i[i])
        b2 = jnp.concatenate([b2r[i], b2i[i]]).reshape(1, 2 * H)
        z = _tc_ffn(z, a2, w1, b1, bm, w2, b2)

    wvec = jnp.zeros((1, 128), f32).at[0, :H].set(out_W[:, 0])
    obvec = jnp.zeros((1, 128), f32).at[0, 0].set(out_b[0])
    out_tile = _tc_final(z, wvec, obvec)
    return out_tile[0:1, 0:1]


# speedup vs baseline: 1.0539x; 1.0539x over previous
import functools
import math

import jax
import jax.numpy as jnp
from jax import lax
from jax.experimental import pallas as pl
from jax.experimental.pallas import tpu as pltpu
from jax.experimental.pallas import tpu_sc as plsc

N = 10000
E = 320000
H = 64
FF = 128
L = 2
ED = 4

N_PAD = 10240
NW = 32
C = 128
NCHUNK = 80
PAIRS = NCHUNK // 2
CW = C * NCHUNK
E_PAD = CW * NW
ROWS_PER_SUB = N_PAD // 16
ACHUNK = E_PAD // 16 // C
C1 = 64
NCHUNK1 = CW // C1
PAIRS1 = NCHUNK1 // 2
NCH_A = 104
NCH_B = 56

BN = 2048


def _mm_bias_kernel(x_ref, w_ref, b_ref, o_ref):
    o_ref[...] = (
        jnp.dot(x_ref[...], w_ref[...], preferred_element_type=jnp.float32, precision=lax.Precision.HIGHEST)
        + b_ref[...]
    )


def _tc_matmul_bias(x, w, b):
    n, k = x.shape
    m = w.shape[1]
    return pl.pallas_call(
        _mm_bias_kernel,
        grid=(n // BN,),
        in_specs=[
            pl.BlockSpec((BN, k), lambda i: (i, 0)),
            pl.BlockSpec((k, m), lambda i: (0, 0)),
            pl.BlockSpec((1, m), lambda i: (0, 0)),
        ],
        out_specs=pl.BlockSpec((BN, m), lambda i: (i, 0)),
        out_shape=jax.ShapeDtypeStruct((n, m), jnp.float32),
    )(x, w, b)


def _qkv_kernel(z_ref, w_ref, q_ref, k_ref, v_ref):
    h = jnp.dot(z_ref[...], w_ref[...], preferred_element_type=jnp.float32, precision=lax.Precision.HIGHEST)
    q_ref[...] = h[:, : 2 * H].astype(jnp.bfloat16)
    k_ref[...] = h[:, 2 * H : 4 * H].astype(jnp.bfloat16)
    v_ref[...] = h[:, 4 * H :]


def _tc_qkv(z, wqkv):
    return pl.pallas_call(
        _qkv_kernel,
        grid=(N_PAD // BN,),
        in_specs=[
            pl.BlockSpec((BN, 2 * H), lambda i: (i, 0)),
            pl.BlockSpec((2 * H, 6 * H), lambda i: (0, 0)),
        ],
        out_specs=[
            pl.BlockSpec((BN, 2 * H), lambda i: (i, 0)),
            pl.BlockSpec((BN, 2 * H), lambda i: (i, 0)),
            pl.BlockSpec((BN, 2 * H), lambda i: (i, 0)),
        ],
        out_shape=[jax.ShapeDtypeStruct((N_PAD, 2 * H), jnp.bfloat16),
                   jax.ShapeDtypeStruct((N_PAD, 2 * H), jnp.bfloat16),
                   jax.ShapeDtypeStruct((N_PAD, 2 * H), jnp.float32)],
    )(z, wqkv)


def _score_kernel(p_ref, ea_ref, b_ref, we_ref, o_ref):
    s = (jnp.dot(p_ref[...], b_ref[...], preferred_element_type=jnp.float32, precision=lax.Precision.HIGHEST)
         + jnp.dot(ea_ref[...], we_ref[...],
                   preferred_element_type=jnp.float32, precision=lax.Precision.HIGHEST))
    o_ref[...] = jnp.exp(s)


def _tc_score(pflat, eaflat, bsel, wesel):
    blk = 4096
    rows = E_PAD // 8
    return pl.pallas_call(
        _score_kernel,
        grid=(rows // blk,),
        in_specs=[
            pl.BlockSpec((blk, 128), lambda i: (i, 0)),
            pl.BlockSpec((blk, 8 * ED), lambda i: (i, 0)),
            pl.BlockSpec((128, 8), lambda i: (0, 0)),
            pl.BlockSpec((8 * ED, 8), lambda i: (0, 0)),
        ],
        out_specs=pl.BlockSpec((blk, 8), lambda i: (i, 0)),
        out_shape=jax.ShapeDtypeStruct((rows, 8), jnp.float32),
    )(pflat, eaflat, bsel, wesel)


def _ffn_kernel(z_ref, a_ref, w1_ref, b1_ref, bm_ref, w2_ref, b2_ref, o_ref):
    za = (z_ref[...] + a_ref[0].astype(jnp.float32)
          + a_ref[1].astype(jnp.float32))
    h = jnp.dot(za, w1_ref[...], preferred_element_type=jnp.float32, precision=lax.Precision.HIGHEST) + b1_ref[...]
    hr = h[:, :FF]
    hi = h[:, FF:]
    mag = jnp.sqrt(hr * hr + hi * hi + 1e-6)
    s = jnp.maximum(mag + bm_ref[...], 0.0) / mag
    hs = jnp.concatenate([hr * s, hi * s], axis=1)
    f = jnp.dot(hs, w2_ref[...], preferred_element_type=jnp.float32, precision=lax.Precision.HIGHEST) + b2_ref[...]
    o_ref[...] = f + za


def _tc_ffn(z, a2, w1, b1, bm, w2, b2):
    return pl.pallas_call(
        _ffn_kernel,
        grid=(N_PAD // BN,),
        in_specs=[
            pl.BlockSpec((BN, 2 * H), lambda i: (i, 0)),
            pl.BlockSpec((2, BN, 2 * H), lambda i: (0, i, 0)),
            pl.BlockSpec((2 * H, 2 * FF), lambda i: (0, 0)),
            pl.BlockSpec((1, 2 * FF), lambda i: (0, 0)),
            pl.BlockSpec((1, FF), lambda i: (0, 0)),
            pl.BlockSpec((2 * FF, 2 * H), lambda i: (0, 0)),
            pl.BlockSpec((1, 2 * H), lambda i: (0, 0)),
        ],
        out_specs=pl.BlockSpec((BN, 2 * H), lambda i: (i, 0)),
        out_shape=jax.ShapeDtypeStruct((N_PAD, 2 * H), jnp.float32),
    )(z, a2, w1, b1, bm, w2, b2)


def _final_kernel(z_ref, wv_ref, ob_ref, o_ref, acc_ref):
    i = pl.program_id(0)

    @pl.when(i == 0)
    def _init():
        acc_ref[...] = jnp.zeros_like(acc_ref)

    z = z_ref[...]
    zr = z[:, :H]
    zi = z[:, H:]
    mz = jnp.sqrt(zr * zr + zi * zi + 1e-6)
    row = i * BN + lax.broadcasted_iota(jnp.int32, (BN, H), 0)
    mz = jnp.where(row < N, mz, 0.0)
    part = jnp.sum(mz, axis=0, keepdims=True)
    partp = jnp.concatenate([part, jnp.zeros((1, H), jnp.float32)], axis=1)
    acc_ref[0:1, :] = acc_ref[0:1, :] + partp

    o_ref[...] = jnp.zeros((8, 128), jnp.float32)

    @pl.when(i == pl.num_programs(0) - 1)
    def _fin():
        tot = jnp.sum(acc_ref[0:1, :] * wv_ref[...])
        outv = tot + float(N) * ob_ref[0, 0]
        ri = lax.broadcasted_iota(jnp.int32, (8, 128), 0)
        ci = lax.broadcasted_iota(jnp.int32, (8, 128), 1)
        o_ref[...] = jnp.where((ri == 0) & (ci == 0), outv, 0.0)


def _tc_final(z, wvec, obvec):
    return pl.pallas_call(
        _final_kernel,
        grid=(N_PAD // BN,),
        in_specs=[
            pl.BlockSpec((BN, 2 * H), lambda i: (i, 0)),
            pl.BlockSpec((1, 128), lambda i: (0, 0)),
            pl.BlockSpec((1, 128), lambda i: (0, 0)),
        ],
        out_specs=pl.BlockSpec((8, 128), lambda i: (0, 0)),
        out_shape=jax.ShapeDtypeStruct((8, 128), jnp.float32),
        scratch_shapes=[pltpu.VMEM((8, 128), jnp.float32)],
    )(z, wvec, obvec)


def _mesh():
    return plsc.VectorSubcoreMesh(
        core_axis_name="c", subcore_axis_name="s", num_cores=2, num_subcores=16
    )


_SC_PARAMS = pltpu.CompilerParams(needs_layout_passes=False)


def _pass1_body(qk_hbm, src_hbm, dst_hbm,
                p_out,
                sidx0, sidx1, didx0, didx1,
                qbuf0, qbuf1, kbuf0, kbuf1, pbuf0, pbuf1,
                qk_sh,
                s_si0, s_si1, s_di0, s_di1, s_q0, s_q1, s_k0, s_k1,
                s_pw0, s_pw1):
    sidx = [sidx0, sidx1]
    didx = [didx0, didx1]
    qbuf = [qbuf0, qbuf1]
    kbuf = [kbuf0, kbuf1]
    pbuf = [pbuf0, pbuf1]
    s_si = [s_si0, s_si1]
    s_di = [s_di0, s_di1]
    s_q = [s_q0, s_q1]
    s_k = [s_k0, s_k1]
    s_pw = [s_pw0, s_pw1]

    cid = lax.axis_index("c")
    sid = lax.axis_index("s")
    wid = sid * 2 + cid
    base0 = wid * CW

    pltpu.sync_copy(qk_hbm.at[pl.ds(sid * ROWS_PER_SUB, ROWS_PER_SUB)],
                    qk_sh.at[pl.ds(sid * ROWS_PER_SUB, ROWS_PER_SUB)])
    plsc.subcore_barrier()

    def issue_idx(t, b):
        base = pl.multiple_of(base0 + t * C1, C1)
        pltpu.async_copy(src_hbm.at[pl.ds(base, C1)], sidx[b], s_si[b])
        pltpu.async_copy(dst_hbm.at[pl.ds(base, C1)], didx[b], s_di[b])

    def wait_idx(b):
        pltpu.make_async_copy(src_hbm.at[pl.ds(0, C1)], sidx[b], s_si[b]).wait()
        pltpu.make_async_copy(dst_hbm.at[pl.ds(0, C1)], didx[b], s_di[b]).wait()

    def issue_gather(b):
        pltpu.async_copy(qk_sh.at[didx[b]], qbuf[b], s_q[b])
        pltpu.async_copy(qk_sh.at[sidx[b]], kbuf[b], s_k[b])

    def wait_gather(b):
        pltpu.make_async_copy(qk_sh.at[didx[b]], qbuf[b], s_q[b]).wait()
        pltpu.make_async_copy(qk_sh.at[sidx[b]], kbuf[b], s_k[b]).wait()

    def compute(t, b):
        @pl.when(t < NCHUNK1 - 2)
        def _pref():
            issue_idx(t + 2, b)

        @pl.when(t >= 2)
        def _wb():
            pltpu.make_async_copy(pbuf[b], p_out.at[pl.ds(0, C1 // 8)],
                                  s_pw[b]).wait()

        qb = qbuf[b]
        kb = kbuf[b]
        pb = pbuf[b]

        def _group(g, gcarry):
            for e in range(16):
                row = g * 16 + e
                acc0 = jnp.zeros((16,), jnp.float32)
                acc1 = jnp.zeros((16,), jnp.float32)
                for j in range(4):
                    qv = plsc.bitcast(qb[row, pl.ds(j * 16, 16)],
                                      jnp.bfloat16)
                    kv = plsc.bitcast(kb[row, pl.ds(64 + j * 16, 16)],
                                      jnp.bfloat16)
                    pa, pbv = plsc.unpack(qv * kv,
                                          format=plsc.PackFormat.INTERLEAVED)
                    acc0 = acc0 + pa
                    acc1 = acc1 + pbv
                pb[2 * g + e // 8, pl.ds((e % 8) * 16, 16)] = acc0 + acc1
            return gcarry

        lax.fori_loop(0, C1 // 16, _group, 0)
        base = pl.multiple_of(base0 + t * C1, C1)
        pltpu.async_copy(
            pb,
            p_out.at[pl.ds(pl.multiple_of(base // 8, C1 // 8), C1 // 8)],
            s_pw[b])

    issue_idx(0, 0)
    issue_idx(1, 1)
    wait_idx(0)
    issue_gather(0)

    def _pair(p, carry):
        t0 = 2 * p
        wait_idx(1)
        issue_gather(1)
        wait_gather(0)
        compute(t0, 0)

        @pl.when(p < PAIRS1 - 1)
        def _nxt():
            wait_idx(0)
            issue_gather(0)

        wait_gather(1)
        compute(t0 + 1, 1)
        return carry

    lax.fori_loop(0, PAIRS1, _pair, 0)
    pltpu.make_async_copy(pbuf[0], p_out.at[pl.ds(0, C1 // 8)], s_pw[0]).wait()
    pltpu.make_async_copy(pbuf[1], p_out.at[pl.ds(0, C1 // 8)], s_pw[1]).wait()


def _sc_pass1(qk, src, dst):
    fn = pl.kernel(
        _pass1_body,
        out_type=[
            jax.ShapeDtypeStruct((E_PAD // 8, 128), jnp.float32),
        ],
        mesh=_mesh(),
        scratch_types=[
            pltpu.VMEM((C1,), jnp.int32),
            pltpu.VMEM((C1,), jnp.int32),
            pltpu.VMEM((C1,), jnp.int32),
            pltpu.VMEM((C1,), jnp.int32),
            pltpu.VMEM((C1, 128), jnp.int32),
            pltpu.VMEM((C1, 128), jnp.int32),
            pltpu.VMEM((C1, 128), jnp.int32),
            pltpu.VMEM((C1, 128), jnp.int32),
            pltpu.VMEM((C1 // 8, 128), jnp.float32),
            pltpu.VMEM((C1 // 8, 128), jnp.float32),
            pltpu.VMEM_SHARED((N_PAD, 128), jnp.int32),
        ] + [pltpu.SemaphoreType.DMA] * 10,
        compiler_params=_SC_PARAMS,
    )
    return fn(qk, src, dst)[0]


def _pass2_body(v_hbm, ex_hbm, src_hbm, dst_hbm,
                a_out,
                sidx0, sidx1, didx0, didx1, sdidx0, sdidx1,
                adx0, adx1, aex0, aex1,
                vbuf0, vbuf1, exbuf0, exbuf1, dnb0, dnb1,
                zbuf, zfbuf, den_sh, a_sh,
                s_si0, s_si1, s_di0, s_di1, s_v0, s_v1, s_x0, s_x1,
                s_d0, s_d1, s_sc0, s_sc1,
                s_ai0, s_ai1, s_ax0, s_ax1, s_as0, s_as1):
    sidx = [sidx0, sidx1]
    didx = [didx0, didx1]
    sdidx = [sdidx0, sdidx1]
    adx = [adx0, adx1]
    aex = [aex0, aex1]
    vbuf = [vbuf0, vbuf1]
    exbuf = [exbuf0, exbuf1]
    dnb = [dnb0, dnb1]
    s_si = [s_si0, s_si1]
    s_di = [s_di0, s_di1]
    s_v = [s_v0, s_v1]
    s_x = [s_x0, s_x1]
    s_d = [s_d0, s_d1]
    s_sc = [s_sc0, s_sc1]
    s_ai = [s_ai0, s_ai1]
    s_ax = [s_ax0, s_ax1]
    s_as = [s_as0, s_as1]

    cid = lax.axis_index("c")
    sid = lax.axis_index("s")
    nch = jnp.where(cid == 0, NCH_A, NCH_B)
    npair = nch // 2
    base0 = sid * (2 * CW) + cid * (NCH_A * C)

    for r in range(16):
        for j in range(8):
            zbuf[r, pl.ds(j * 16, 16)] = jnp.zeros((16,), jnp.float32)
    for j in range(8):
        zfbuf[pl.ds(j * 16, 16)] = jnp.zeros((16,), jnp.float32)

    def _zrow(r, carry):
        pltpu.sync_copy(zbuf, a_sh.at[pl.ds(sid * ROWS_PER_SUB + r * 16, 16)])
        return carry

    lax.fori_loop(0, ROWS_PER_SUB // 16, _zrow, 0)

    def _zden(r, carry):
        pltpu.sync_copy(zfbuf,
                        den_sh.at[pl.ds(sid * ROWS_PER_SUB + r * 128, 128)])
        return carry

    lax.fori_loop(0, ROWS_PER_SUB // 128, _zden, 0)
    plsc.subcore_barrier()

    abase0 = sid * (E_PAD // 16)

    def a_issue_idx(t, b):
        base = pl.multiple_of(abase0 + t * C, C)
        pltpu.async_copy(dst_hbm.at[pl.ds(base, C)], adx[b], s_ai[b])
        pltpu.async_copy(ex_hbm.at[pl.ds(base, C)], aex[b], s_ax[b])

    def a_wait_idx(b):
        pltpu.make_async_copy(dst_hbm.at[pl.ds(0, C)], adx[b], s_ai[b]).wait()
        pltpu.make_async_copy(ex_hbm.at[pl.ds(0, C)], aex[b], s_ax[b]).wait()

    def a_issue_sc(b):
        pltpu.async_copy(aex[b], den_sh.at[adx[b]], s_as[b], add=True)

    def a_wait_sc(b):
        pltpu.make_async_copy(aex[b], den_sh.at[adx[b]], s_as[b]).wait()

    a_issue_idx(0, 0)
    a_issue_idx(1, 1)

    def _apair(p, carry):
        t0 = 2 * p
        a_wait_idx(0)
        a_issue_sc(0)
        a_wait_idx(1)
        a_issue_sc(1)
        a_wait_sc(0)

        @pl.when(p < ACHUNK // 2 - 1)
        def _n0():
            a_issue_idx(t0 + 2, 0)

        a_wait_sc(1)

        @pl.when(p < ACHUNK // 2 - 1)
        def _n1():
            a_issue_idx(t0 + 3, 1)

        return carry

    lax.fori_loop(0, ACHUNK // 2, _apair, 0)
    plsc.subcore_barrier()

    def issue_idx(t, b):
        base = pl.multiple_of(base0 + t * C, C)
        pltpu.async_copy(src_hbm.at[pl.ds(base, C)], sidx[b], s_si[b])
        pltpu.async_copy(dst_hbm.at[pl.ds(base, C)], didx[b], s_di[b])

    def wait_idx(b):
        pltpu.make_async_copy(src_hbm.at[pl.ds(0, C)], sidx[b], s_si[b]).wait()
        pltpu.make_async_copy(dst_hbm.at[pl.ds(0, C)], didx[b], s_di[b]).wait()

    def issue_gather(t, b):
        base = pl.multiple_of(base0 + t * C, C)
        pltpu.async_copy(v_hbm.at[sidx[b]], vbuf[b], s_v[b])
        pltpu.async_copy(ex_hbm.at[pl.ds(base, C)], exbuf[b], s_x[b])
        pltpu.async_copy(den_sh.at[didx[b]], dnb[b], s_d[b])

    def wait_gather(b):
        pltpu.make_async_copy(v_hbm.at[sidx[b]], vbuf[b], s_v[b]).wait()
        pltpu.make_async_copy(ex_hbm.at[pl.ds(0, C)], exbuf[b], s_x[b]).wait()
        pltpu.make_async_copy(den_sh.at[didx[b]], dnb[b], s_d[b]).wait()

    def issue_scatter(b):
        pltpu.async_copy(vbuf[b], a_sh.at[sdidx[b]], s_sc[b], add=True)

    def wait_scatter(b):
        pltpu.make_async_copy(vbuf[b], a_sh.at[sdidx[b]], s_sc[b]).wait()

    def compute(t, b):
        for gg in range(C // 16):
            sdidx[b][pl.ds(gg * 16, 16)] = didx[b][pl.ds(gg * 16, 16)]

        @pl.when(t < nch - 2)
        def _pref():
            issue_idx(t + 2, b)

        vb = vbuf[b]
        exb = exbuf[b]
        dn = dnb[b]

        def _group(g, gcarry):
            ex16 = exb[pl.ds(g * 16, 16)]
            den16 = dn[pl.ds(g * 16, 16)]
            alpha16 = ex16 / (den16 + 1e-9)
            for e in range(16):
                row = g * 16 + e
                av = jnp.full((16,), alpha16[e])
                for j in range(8):
                    vb[row, pl.ds(j * 16, 16)] = (
                        vb[row, pl.ds(j * 16, 16)] * av
                    )
            return gcarry

        lax.fori_loop(0, C // 16, _group, 0)

    issue_idx(0, 0)
    issue_idx(1, 1)
    wait_idx(0)
    issue_gather(0, 0)

    def _pair(p, carry):
        t0 = 2 * p

        @pl.when(p > 0)
        def _ws1():
            wait_scatter(1)

        wait_idx(1)
        issue_gather(t0 + 1, 1)
        wait_gather(0)
        compute(t0, 0)
        issue_scatter(0)

        @pl.when(p < npair - 1)
        def _nxt():
            wait_scatter(0)
            wait_idx(0)
            issue_gather(t0 + 2, 0)

        wait_gather(1)
        compute(t0 + 1, 1)
        issue_scatter(1)
        return carry

    lax.fori_loop(0, npair, _pair, 0)
    wait_scatter(0)
    wait_scatter(1)
    plsc.subcore_barrier()
    pltpu.sync_copy(
        a_sh.at[pl.ds(sid * ROWS_PER_SUB, ROWS_PER_SUB)],
        a_out.at[cid, pl.ds(sid * ROWS_PER_SUB, ROWS_PER_SUB)],
    )


def _sc_pass2(v, ex, src, dst):
    fn = pl.kernel(
        _pass2_body,
        out_type=[
            jax.ShapeDtypeStruct((2, N_PAD, 2 * H), jnp.float32),
        ],
        mesh=_mesh(),
        scratch_types=[
            pltpu.VMEM((C,), jnp.int32),
            pltpu.VMEM((C,), jnp.int32),
            pltpu.VMEM((C,), jnp.int32),
            pltpu.VMEM((C,), jnp.int32),
            pltpu.VMEM((C,), jnp.int32),
            pltpu.VMEM((C,), jnp.int32),
            pltpu.VMEM((C,), jnp.int32),
            pltpu.VMEM((C,), jnp.int32),
            pltpu.VMEM((C,), jnp.float32),
            pltpu.VMEM((C,), jnp.float32),
            pltpu.VMEM((C, 2 * H), jnp.float32),
            pltpu.VMEM((C, 2 * H), jnp.float32),
            pltpu.VMEM((C,), jnp.float32),
            pltpu.VMEM((C,), jnp.float32),
            pltpu.VMEM((C,), jnp.float32),
            pltpu.VMEM((C,), jnp.float32),
            pltpu.VMEM((16, 2 * H), jnp.float32),
            pltpu.VMEM((128,), jnp.float32),
            pltpu.VMEM_SHARED((N_PAD,), jnp.float32),
            pltpu.VMEM_SHARED((N_PAD, 2 * H), jnp.float32),
        ] + [pltpu.SemaphoreType.DMA] * 18,
        compiler_params=_SC_PARAMS,
    )
    return fn(v, ex, src, dst)[0]


def _cplx(wr, wi):
    return jnp.concatenate(
        [jnp.concatenate([wr, wi], axis=1),
         jnp.concatenate([-wi, wr], axis=1)],
        axis=0,
    )


def kernel(atom_types, coords_spherical, edge_index, edge_attr,
           emb_Wr, emb_Wi, emb_br, emb_bi,
           Wq_r, Wq_i, Wk_r, Wk_i, Wv_r, Wv_i, we,
           W1r, W1i, b1r, b1i, b_mod, W2r, W2i, b2r, b2i,
           out_W, out_b):
    f32 = jnp.float32
    d_in = atom_types.shape[1] + 3
    k_pad = 256

    x = jnp.concatenate([atom_types, coords_spherical], axis=1)
    x_pad = jnp.zeros((N_PAD, k_pad), f32).at[:N, :d_in].set(x)
    w_emb = jnp.zeros((k_pad, 2 * H), f32).at[:d_in].set(
        jnp.concatenate([emb_Wr, emb_Wi], axis=1))
    b_emb = jnp.concatenate([emb_br, emb_bi]).reshape(1, 2 * H)

    pad_e = E_PAD - E
    pad_idx = jnp.full((pad_e,), N_PAD - 1, jnp.int32)
    src_pad = jnp.concatenate([edge_index[0].astype(jnp.int32), pad_idx])
    dst_pad = jnp.concatenate([edge_index[1].astype(jnp.int32), pad_idx])
    ea_flat = jnp.concatenate(
        [edge_attr.astype(f32), jnp.zeros((pad_e, ED), f32)], axis=0
    ).reshape(E_PAD // 8, 8 * ED)

    bsel = (lax.broadcasted_iota(jnp.int32, (128, 8), 0) // 16
            == lax.broadcasted_iota(jnp.int32, (128, 8), 1)).astype(f32)

    z = _tc_matmul_bias(x_pad, w_emb, b_emb)

    inv_sqrt_h = 1.0 / math.sqrt(float(H))
    for i in range(L):
        wqkv = jnp.concatenate(
            [_cplx(Wq_r[i], Wq_i[i]) * inv_sqrt_h,
             _cplx(Wk_r[i], Wk_i[i]),
             _cplx(Wv_r[i], Wv_i[i])],
            axis=1,
        )
        q, k, v = _tc_qkv(z, wqkv)
        qk32 = lax.bitcast_convert_type(
            jnp.concatenate([q, k], axis=1).reshape(N_PAD, 128, 2), jnp.int32)

        wesel = ((lax.broadcasted_iota(jnp.int32, (8 * ED, 8), 0) // ED
                  == lax.broadcasted_iota(jnp.int32, (8 * ED, 8), 1))
                 .astype(f32)
                 * jnp.tile(we[i].astype(f32), 8)[:, None])

        pflat = _sc_pass1(qk32, src_pad, dst_pad)
        ex = _tc_score(pflat, ea_flat, bsel, wesel).reshape(E_PAD)
        a2 = _sc_pass2(v, ex, src_pad, dst_pad)

        w1 = _cplx(W1r[i], W1i[i])
        b1 = jnp.concatenate([b1r[i], b1i[i]]).reshape(1, 2 * FF)
        bm = b_mod[i].reshape(1, FF)
        w2 = _cplx(W2r[i], W2i[i])
        b2 = jnp.concatenate([b2r[i], b2i[i]]).reshape(1, 2 * H)
        z = _tc_ffn(z, a2, w1, b1, bm, w2, b2)

    wvec = jnp.zeros((1, 128), f32).at[0, :H].set(out_W[:, 0])
    obvec = jnp.zeros((1, 128), f32).at[0, 0].set(out_b[0])
    out_tile = _tc_final(z, wvec, obvec)
    return out_tile[0:1, 0:1]
